# Initial kernel scaffold; baseline (speedup 1.0000x reference)
import jax
import jax.numpy as jnp
from jax.experimental import pallas as pl


def kernel(node_features, edge_index, W1, b1, W2, b2, W3, b3, W4, b4, W5, b5, W6, b6):
    raise NotImplementedError("write your pallas kernel here")


# speedup vs baseline: 1.1813x; 1.1813x over previous
import functools

import jax
import jax.numpy as jnp
from jax import lax
from jax.experimental import pallas as pl
from jax.experimental.pallas import tpu as pltpu
from jax.experimental.pallas import tpu_sc as plsc

_NUM_CORES = 2
_NUM_SUBCORES = 16
_NW = _NUM_CORES * _NUM_SUBCORES


def _node_projection(x, w1a, w1b, b1):
    n, d = x.shape
    h = w1a.shape[1]

    def body(x_ref, wa_ref, wb_ref, b1_ref, t_ref):
        xv = x_ref[...]
        t_ref[0:n, :] = (
            jnp.dot(xv, wa_ref[...], preferred_element_type=jnp.float32)
            + b1_ref[...]
        )
        t_ref[n : 2 * n, :] = jnp.dot(
            xv, wb_ref[...], preferred_element_type=jnp.float32
        )

    return pl.pallas_call(
        body,
        out_shape=jax.ShapeDtypeStruct((2 * n, h), jnp.float32),
    )(x, w1a, w1b, b1.reshape(1, h))


def _sc_gather(table, j3, nchunk, chunk):
    rows_total = _NW * nchunk * chunk
    h = table.shape[1]
    epw = nchunk * chunk

    mesh = plsc.VectorSubcoreMesh(core_axis_name="c", subcore_axis_name="s")

    @functools.partial(
        pl.kernel,
        mesh=mesh,
        out_type=jax.ShapeDtypeStruct((rows_total, h), jnp.float32),
        scratch_types=[
            pltpu.VMEM((nchunk, chunk), jnp.int32),
            pltpu.VMEM((chunk, h), jnp.float32),
            pltpu.SemaphoreType.DMA,
        ],
    )
    def gather_kernel(t_hbm, j_hbm, g_hbm, idx_v, rows_v, sem):
        wid = lax.axis_index("s") * _NUM_CORES + lax.axis_index("c")
        pltpu.sync_copy(j_hbm.at[wid], idx_v)
        base = wid * epw

        def body(i, carry):
            pltpu.async_copy(t_hbm.at[idx_v.at[i]], rows_v, sem).wait()
            pltpu.sync_copy(rows_v, g_hbm.at[pl.ds(base + i * chunk, chunk)])
            return carry

        lax.fori_loop(0, nchunk, body, 0)

    return gather_kernel(table, j3)


def _edge_mlp(gp, w2, b2, w35, b35, w46, b8, block_e):
    e, two_h = gp.shape
    h = two_h // 2
    f = w2.shape[1]
    grid = (e // block_e,)

    def body(g_ref, w2_ref, b2_ref, w35_ref, b35_ref, w46_ref, b8_ref,
             ef_ref, o8_ref):
        g = g_ref[...]
        hid = jnp.maximum(g[:, :h] + g[:, h:], 0.0)
        ef = (
            jnp.dot(hid, w2_ref[...], preferred_element_type=jnp.float32)
            + b2_ref[...]
        )
        ef_ref[...] = ef
        u = jnp.maximum(
            jnp.dot(ef, w35_ref[...], preferred_element_type=jnp.float32)
            + b35_ref[...],
            0.0,
        )
        o8_ref[...] = (
            jnp.dot(u, w46_ref[...], preferred_element_type=jnp.float32)
            + b8_ref[...]
        )

    full = lambda shape: pl.BlockSpec(shape, lambda i: (0, 0))
    return pl.pallas_call(
        body,
        grid=grid,
        in_specs=[
            pl.BlockSpec((block_e, two_h), lambda i: (i, 0)),
            full(w2.shape),
            full((1, f)),
            full(w35.shape),
            full((1, w35.shape[1])),
            full(w46.shape),
            full((1, 8)),
        ],
        out_specs=[
            pl.BlockSpec((block_e, f), lambda i: (i, 0)),
            pl.BlockSpec((block_e, 8), lambda i: (i, 0)),
        ],
        out_shape=[
            jax.ShapeDtypeStruct((e, f), jnp.float32),
            jax.ShapeDtypeStruct((e, 8), jnp.float32),
        ],
    )(gp, w2, b2.reshape(1, f), w35, b35.reshape(1, -1), w46,
      b8.reshape(1, 8))


def kernel(node_features, edge_index, W1, b1, W2, b2, W3, b3, W4, b4,
           W5, b5, W6, b6):
    n, d = node_features.shape
    e = edge_index.shape[1]
    hdim = W1.shape[1]
    f = W2.shape[1]
    c = W6.shape[1]

    table = _node_projection(node_features, W1[:d], W1[d:], b1)

    j = jnp.stack([edge_index[0], edge_index[1] + n], axis=1).reshape(-1)
    epw = (2 * e) // _NW
    chunk = next(cc for cc in range(128, 7, -8) if epw % cc == 0)
    nchunk = epw // chunk
    j3 = j.reshape(_NW, nchunk, chunk)

    g = _sc_gather(table, j3, nchunk, chunk)
    gp = g.reshape(e, 2 * hdim)

    w35 = jnp.concatenate([W3, W5], axis=1)
    b35 = jnp.concatenate([b3, b5])
    w46 = jnp.zeros((2 * hdim, 8), jnp.float32)
    w46 = w46.at[:hdim, 0:1].set(W4).at[hdim:, 1 : 1 + c].set(W6)
    b8 = jnp.zeros((8,), jnp.float32).at[0].set(b4[0]).at[1 : 1 + c].set(b6)

    block_e = 2560
    ef, o8 = _edge_mlp(gp, W2, b2, w35, b35, w46, b8, block_e)

    existence = o8[:, 0]
    assignment = o8[:, 1 : 1 + c]
    return (existence, assignment, ef)


# speedup vs baseline: 2.1443x; 1.8151x over previous
import functools

import jax
import jax.numpy as jnp
from jax import lax
from jax.experimental import pallas as pl
from jax.experimental.pallas import tpu as pltpu
from jax.experimental.pallas import tpu_sc as plsc

_NUM_CORES = 2
_NUM_SUBCORES = 16
_NW = _NUM_CORES * _NUM_SUBCORES


def _node_projection(x, w1a, w1b, b1):
    n, d = x.shape
    h = w1a.shape[1]

    def body(x_ref, wa_ref, wb_ref, b1_ref, t_ref):
        xv = x_ref[...]
        t_ref[0:n, :] = (
            jnp.dot(xv, wa_ref[...], preferred_element_type=jnp.float32)
            + b1_ref[...]
        )
        t_ref[n : 2 * n, :] = jnp.dot(
            xv, wb_ref[...], preferred_element_type=jnp.float32
        )

    return pl.pallas_call(
        body,
        out_shape=jax.ShapeDtypeStruct((2 * n, h), jnp.float32),
    )(x, w1a, w1b, b1.reshape(1, h))


def _sc_gather(table, j3, nsc, k, chunk):
    rows_total = _NW * nsc * k * chunk
    h = table.shape[1]
    epw = nsc * k * chunk
    scrows = k * chunk

    mesh = plsc.VectorSubcoreMesh(core_axis_name="c", subcore_axis_name="s")

    @functools.partial(
        pl.kernel,
        mesh=mesh,
        out_type=jax.ShapeDtypeStruct((rows_total, h), jnp.float32),
        scratch_types=[
            pltpu.VMEM((nsc * k, chunk), jnp.int32),
            pltpu.VMEM((2, scrows, h), jnp.float32),
            pltpu.SemaphoreType.DMA,
            pltpu.SemaphoreType.DMA,
            pltpu.SemaphoreType.DMA,
            pltpu.SemaphoreType.DMA,
        ],
    )
    def gather_kernel(t_hbm, j_hbm, g_hbm, idx_v, rows_v, g0, g1, w0, w1):
        gsem = (g0, g1)
        wsem = (w0, w1)
        wid = lax.axis_index("s") * _NUM_CORES + lax.axis_index("c")
        pltpu.sync_copy(j_hbm.at[wid], idx_v)
        base = wid * epw

        def issue(sc_idx, slot):
            for q in range(k):
                pltpu.async_copy(
                    t_hbm.at[idx_v.at[sc_idx * k + q]],
                    rows_v.at[slot].at[pl.ds(q * chunk, chunk)],
                    gsem[slot],
                )

        issue(0, 0)
        issue(1, 1)

        def outer(i0, carry):
            for b in range(2):
                i = i0 * 2 + b
                pltpu.make_async_copy(
                    t_hbm.at[pl.ds(0, scrows)], rows_v.at[b], gsem[b]
                ).wait()
                pltpu.async_copy(
                    rows_v.at[b],
                    g_hbm.at[pl.ds(base + i * scrows, scrows)],
                    wsem[b],
                )

                @pl.when(i + 2 < nsc)
                def _():
                    pltpu.make_async_copy(
                        t_hbm.at[pl.ds(0, scrows)], rows_v.at[b], wsem[b]
                    ).wait()
                    issue(i + 2, b)

            return carry

        lax.fori_loop(0, nsc // 2, outer, 0)
        pltpu.make_async_copy(
            t_hbm.at[pl.ds(0, scrows)], rows_v.at[0], wsem[0]
        ).wait()
        pltpu.make_async_copy(
            t_hbm.at[pl.ds(0, scrows)], rows_v.at[1], wsem[1]
        ).wait()

    return gather_kernel(table, j3)


def _edge_mlp(g, e, w2, b2, w35, b35, w46, b8, block_e):
    h = g.shape[1]
    f = w2.shape[1]
    grid = (e // block_e,)
    nblk = e // block_e

    def body(gs_ref, gd_ref, w2_ref, b2_ref, w35_ref, b35_ref, w46_ref,
             b8_ref, ef_ref, o8_ref):
        hid = jnp.maximum(gs_ref[...] + gd_ref[...], 0.0)
        ef = (
            jnp.dot(hid, w2_ref[...], preferred_element_type=jnp.float32)
            + b2_ref[...]
        )
        ef_ref[...] = ef
        u = jnp.maximum(
            jnp.dot(ef, w35_ref[...], preferred_element_type=jnp.float32)
            + b35_ref[...],
            0.0,
        )
        o8_ref[...] = (
            jnp.dot(u, w46_ref[...], preferred_element_type=jnp.float32)
            + b8_ref[...]
        )

    full = lambda shape: pl.BlockSpec(shape, lambda i: (0, 0))
    return pl.pallas_call(
        body,
        grid=grid,
        in_specs=[
            pl.BlockSpec((block_e, h), lambda i: (i, 0)),
            pl.BlockSpec((block_e, h), lambda i: (i + nblk, 0)),
            full(w2.shape),
            full((1, f)),
            full(w35.shape),
            full((1, w35.shape[1])),
            full(w46.shape),
            full((1, 8)),
        ],
        out_specs=[
            pl.BlockSpec((block_e, f), lambda i: (i, 0)),
            pl.BlockSpec((block_e, 8), lambda i: (i, 0)),
        ],
        out_shape=[
            jax.ShapeDtypeStruct((e, f), jnp.float32),
            jax.ShapeDtypeStruct((e, 8), jnp.float32),
        ],
    )(g, g, w2, b2.reshape(1, f), w35, b35.reshape(1, -1), w46,
      b8.reshape(1, 8))


def kernel(node_features, edge_index, W1, b1, W2, b2, W3, b3, W4, b4,
           W5, b5, W6, b6):
    n, d = node_features.shape
    e = edge_index.shape[1]
    hdim = W1.shape[1]
    f = W2.shape[1]
    c = W6.shape[1]

    table = _node_projection(node_features, W1[:d], W1[d:], b1)

    j = jnp.concatenate([edge_index[0], edge_index[1] + n])
    epw = (2 * e) // _NW
    chunk = next(cc for cc in range(64, 7, -8) if epw % cc == 0)
    nchunk = epw // chunk
    k = next(kk for kk in range(5, 0, -1) if nchunk % (2 * kk) == 0)
    nsc = nchunk // k
    j3 = j.reshape(_NW, nchunk, chunk)

    g = _sc_gather(table, j3, nsc, k, chunk)

    w35 = jnp.concatenate([W3, W5], axis=1)
    b35 = jnp.concatenate([b3, b5])
    w46 = jnp.zeros((2 * hdim, 8), jnp.float32)
    w46 = w46.at[:hdim, 0:1].set(W4).at[hdim:, 1 : 1 + c].set(W6)
    b8 = jnp.zeros((8,), jnp.float32).at[0].set(b4[0]).at[1 : 1 + c].set(b6)

    block_e = 2560
    ef, o8 = _edge_mlp(g, e, W2, b2, w35, b35, w46, b8, block_e)

    existence = o8[:, 0]
    assignment = o8[:, 1 : 1 + c]
    return (existence, assignment, ef)


# speedup vs baseline: 3.7804x; 1.7630x over previous
import functools

import jax
import jax.numpy as jnp
from jax import lax
from jax.experimental import pallas as pl
from jax.experimental.pallas import tpu as pltpu
from jax.experimental.pallas import tpu_sc as plsc

_NUM_CORES = 2
_NUM_SUBCORES = 16
_NW = _NUM_CORES * _NUM_SUBCORES


def _node_projection(x, w1a, w1b, b1):
    n, d = x.shape
    h = w1a.shape[1]

    def body(x_ref, wa_ref, wb_ref, b1_ref, t_ref):
        xv = x_ref[...]
        t_ref[0:n, :] = (
            jnp.dot(xv, wa_ref[...], preferred_element_type=jnp.float32)
            + b1_ref[...]
        )
        t_ref[n : 2 * n, :] = jnp.dot(
            xv, wb_ref[...], preferred_element_type=jnp.float32
        )

    return pl.pallas_call(
        body,
        out_shape=jax.ShapeDtypeStruct((2 * n, h), jnp.float32),
    )(x, w1a, w1b, b1.reshape(1, h))


def _sc_gather(table, j3, nsc, k, chunk):
    rows_total = _NW * nsc * k * chunk
    h = table.shape[1]
    epw = nsc * k * chunk
    scrows = k * chunk

    mesh = plsc.VectorSubcoreMesh(core_axis_name="c", subcore_axis_name="s")

    @functools.partial(
        pl.kernel,
        mesh=mesh,
        out_type=jax.ShapeDtypeStruct((rows_total, h), jnp.float32),
        scratch_types=[
            pltpu.VMEM((nsc * k, chunk), jnp.int32),
            pltpu.VMEM((2, scrows, h), jnp.float32),
            pltpu.SemaphoreType.DMA,
            pltpu.SemaphoreType.DMA,
            pltpu.SemaphoreType.DMA,
            pltpu.SemaphoreType.DMA,
        ],
    )
    def gather_kernel(t_hbm, j_hbm, g_hbm, idx_v, rows_v, g0, g1, w0, w1):
        gsem = (g0, g1)
        wsem = (w0, w1)
        wid = lax.axis_index("s") * _NUM_CORES + lax.axis_index("c")
        pltpu.sync_copy(j_hbm.at[wid], idx_v)
        base = wid * epw

        def issue(sc_idx, slot):
            for q in range(k):
                pltpu.async_copy(
                    t_hbm.at[idx_v.at[sc_idx * k + q]],
                    rows_v.at[slot].at[pl.ds(q * chunk, chunk)],
                    gsem[slot],
                )

        issue(0, 0)
        issue(1, 1)

        def outer(i0, carry):
            for b in range(2):
                i = i0 * 2 + b
                pltpu.make_async_copy(
                    t_hbm.at[pl.ds(0, scrows)], rows_v.at[b], gsem[b]
                ).wait()
                pltpu.async_copy(
                    rows_v.at[b],
                    g_hbm.at[pl.ds(base + i * scrows, scrows)],
                    wsem[b],
                )

                @pl.when(i + 2 < nsc)
                def _():
                    pltpu.make_async_copy(
                        t_hbm.at[pl.ds(0, scrows)], rows_v.at[b], wsem[b]
                    ).wait()
                    issue(i + 2, b)

            return carry

        lax.fori_loop(0, nsc // 2, outer, 0)
        pltpu.make_async_copy(
            t_hbm.at[pl.ds(0, scrows)], rows_v.at[0], wsem[0]
        ).wait()
        pltpu.make_async_copy(
            t_hbm.at[pl.ds(0, scrows)], rows_v.at[1], wsem[1]
        ).wait()

    return gather_kernel(table, j3)


def _edge_mlp(g, e, w2, b2, w35, b35, w46, b8, block_e):
    h = g.shape[1]
    f = w2.shape[1]
    grid = (e // block_e,)
    nblk = e // block_e
    dn_rt = (((1,), (1,)), ((), ()))

    def body(gs_ref, gd_ref, w2t_ref, b2_ref, w35t_ref, b35_ref, w46t_ref,
             b8_ref, eft_ref, o8t_ref):
        hid = jnp.maximum(gs_ref[...] + gd_ref[...], 0.0)
        ef_t = (
            lax.dot_general(w2t_ref[...], hid, dn_rt,
                            preferred_element_type=jnp.float32)
            + b2_ref[...]
        )
        eft_ref[...] = ef_t
        u_t = jnp.maximum(
            lax.dot_general(w35t_ref[...], ef_t,
                            (((1,), (0,)), ((), ())),
                            preferred_element_type=jnp.float32)
            + b35_ref[...],
            0.0,
        )
        o8t_ref[...] = (
            lax.dot_general(w46t_ref[...], u_t,
                            (((1,), (0,)), ((), ())),
                            preferred_element_type=jnp.float32)
            + b8_ref[...]
        )

    full = lambda shape: pl.BlockSpec(shape, lambda i: (0, 0))
    return pl.pallas_call(
        body,
        grid=grid,
        in_specs=[
            pl.BlockSpec((block_e, h), lambda i: (i, 0)),
            pl.BlockSpec((block_e, h), lambda i: (i + nblk, 0)),
            full((f, h)),
            full((f, 1)),
            full((2 * h, f)),
            full((2 * h, 1)),
            full((8, 2 * h)),
            full((8, 1)),
        ],
        out_specs=[
            pl.BlockSpec((f, block_e), lambda i: (0, i)),
            pl.BlockSpec((8, block_e), lambda i: (0, i)),
        ],
        out_shape=[
            jax.ShapeDtypeStruct((f, e), jnp.float32),
            jax.ShapeDtypeStruct((8, e), jnp.float32),
        ],
    )(g, g, w2.T, b2.reshape(f, 1), w35.T, b35.reshape(-1, 1), w46.T,
      b8.reshape(8, 1))


def kernel(node_features, edge_index, W1, b1, W2, b2, W3, b3, W4, b4,
           W5, b5, W6, b6):
    n, d = node_features.shape
    e = edge_index.shape[1]
    hdim = W1.shape[1]
    f = W2.shape[1]
    c = W6.shape[1]

    table = _node_projection(node_features, W1[:d], W1[d:], b1)

    j = jnp.concatenate([edge_index[0], edge_index[1] + n])
    epw = (2 * e) // _NW
    chunk, k = next(
        (cc, kk)
        for cc in range(64, 7, -8)
        for kk in (5, 4, 3, 2, 1)
        if epw % cc == 0 and (epw // cc) % (2 * kk) == 0
    )
    nchunk = epw // chunk
    nsc = nchunk // k
    j3 = j.reshape(_NW, nchunk, chunk)

    g = _sc_gather(table, j3, nsc, k, chunk)

    w35 = jnp.concatenate([W3, W5], axis=1)
    b35 = jnp.concatenate([b3, b5])
    w46 = jnp.zeros((2 * hdim, 8), jnp.float32)
    w46 = w46.at[:hdim, 0:1].set(W4).at[hdim:, 1 : 1 + c].set(W6)
    b8 = jnp.zeros((8,), jnp.float32).at[0].set(b4[0]).at[1 : 1 + c].set(b6)

    block_e = 2560
    ef_t, o8_t = _edge_mlp(g, e, W2, b2, w35, b35, w46, b8, block_e)

    existence = o8_t[0]
    assignment = o8_t[1 : 1 + c].T
    return (existence, assignment, ef_t.T)


# speedup vs baseline: 4.2244x; 1.1174x over previous
import functools

import jax
import jax.numpy as jnp
from jax import lax
from jax.experimental import pallas as pl
from jax.experimental.pallas import tpu as pltpu
from jax.experimental.pallas import tpu_sc as plsc

_NUM_CORES = 2
_NUM_SUBCORES = 16
_NW = _NUM_CORES * _NUM_SUBCORES


def _node_projection(x, w1a, w1b, b1):
    n, d = x.shape
    h = w1a.shape[1]

    def body(x_ref, wa_ref, wb_ref, b1_ref, t_ref):
        xv = x_ref[...]
        t_ref[0:n, :] = (
            jnp.dot(xv, wa_ref[...], preferred_element_type=jnp.float32)
            + b1_ref[...]
        )
        t_ref[n : 2 * n, :] = jnp.dot(
            xv, wb_ref[...], preferred_element_type=jnp.float32
        )

    return pl.pallas_call(
        body,
        out_shape=jax.ShapeDtypeStruct((2 * n, h), jnp.float32),
    )(x, w1a, w1b, b1.reshape(1, h))


def _sc_gather_add(table, j3, nsc, k, chunk):
    h = table.shape[1]
    epw = nsc * k * chunk
    rows_total = _NW * epw
    scrows = k * chunk

    mesh = plsc.VectorSubcoreMesh(core_axis_name="c", subcore_axis_name="s")

    @functools.partial(
        pl.kernel,
        mesh=mesh,
        out_type=jax.ShapeDtypeStruct((rows_total, h), jnp.float32),
        scratch_types=[
            pltpu.VMEM((2 * nsc * k, chunk), jnp.int32),
            pltpu.VMEM((2, scrows, h), jnp.float32),
            pltpu.SemaphoreType.DMA,
            pltpu.SemaphoreType.DMA,
            pltpu.SemaphoreType.DMA,
            pltpu.SemaphoreType.DMA,
        ],
    )
    def gather_kernel(t_hbm, j_hbm, g_hbm, idx_v, rows_v, g0, g1, w0, w1):
        gsem = (g0, g1)
        wsem = (w0, w1)
        wid = lax.axis_index("s") * _NUM_CORES + lax.axis_index("c")
        pltpu.sync_copy(j_hbm.at[wid], idx_v)
        base = wid * epw

        def issue(sc_idx, slot, phase):
            for q in range(k):
                ci = (sc_idx * k + q) * 2 + phase
                pltpu.async_copy(
                    t_hbm.at[idx_v.at[ci]],
                    rows_v.at[slot].at[pl.ds(q * chunk, chunk)],
                    gsem[slot],
                    add=(phase == 1),
                )

        def drain(sem, slot):
            pltpu.make_async_copy(
                t_hbm.at[pl.ds(0, scrows)], rows_v.at[slot], sem
            ).wait()

        issue(0, 0, 0)
        issue(1, 1, 0)

        def outer(i0, carry):
            for b in range(2):
                i = i0 * 2 + b
                drain(gsem[b], b)
                issue(i, b, 1)
                drain(gsem[b], b)
                pltpu.async_copy(
                    rows_v.at[b],
                    g_hbm.at[pl.ds(base + i * scrows, scrows)],
                    wsem[b],
                )

                @pl.when(i + 2 < nsc)
                def _():
                    drain(wsem[b], b)
                    issue(i + 2, b, 0)

            return carry

        lax.fori_loop(0, nsc // 2, outer, 0)
        drain(wsem[0], 0)
        drain(wsem[1], 1)

    return gather_kernel(table, j3)


def _edge_mlp(g, w2, b2, w35, b35, w46, b8, block_e):
    e, h = g.shape
    f = w2.shape[1]
    grid = (e // block_e,)
    dn_rt = (((1,), (1,)), ((), ()))
    dn_std = (((1,), (0,)), ((), ()))

    def body(g_ref, w2t_ref, b2_ref, w35t_ref, b35_ref, w46t_ref,
             b8_ref, eft_ref, o8t_ref):
        hid = jnp.maximum(g_ref[...], 0.0)
        ef_t = (
            lax.dot_general(w2t_ref[...], hid, dn_rt,
                            preferred_element_type=jnp.float32)
            + b2_ref[...]
        )
        eft_ref[...] = ef_t
        u_t = jnp.maximum(
            lax.dot_general(w35t_ref[...], ef_t, dn_std,
                            preferred_element_type=jnp.float32)
            + b35_ref[...],
            0.0,
        )
        o8t_ref[...] = (
            lax.dot_general(w46t_ref[...], u_t, dn_std,
                            preferred_element_type=jnp.float32)
            + b8_ref[...]
        )

    full = lambda shape: pl.BlockSpec(shape, lambda i: (0, 0))
    return pl.pallas_call(
        body,
        grid=grid,
        in_specs=[
            pl.BlockSpec((block_e, h), lambda i: (i, 0)),
            full((f, h)),
            full((f, 1)),
            full((2 * h, f)),
            full((2 * h, 1)),
            full((8, 2 * h)),
            full((8, 1)),
        ],
        out_specs=[
            pl.BlockSpec((f, block_e), lambda i: (0, i)),
            pl.BlockSpec((8, block_e), lambda i: (0, i)),
        ],
        out_shape=[
            jax.ShapeDtypeStruct((f, e), jnp.float32),
            jax.ShapeDtypeStruct((8, e), jnp.float32),
        ],
    )(g, w2.T, b2.reshape(f, 1), w35.T, b35.reshape(-1, 1), w46.T,
      b8.reshape(8, 1))


def kernel(node_features, edge_index, W1, b1, W2, b2, W3, b3, W4, b4,
           W5, b5, W6, b6):
    n, d = node_features.shape
    e = edge_index.shape[1]
    hdim = W1.shape[1]
    f = W2.shape[1]
    c = W6.shape[1]

    table = _node_projection(node_features, W1[:d], W1[d:], b1)

    epw = e // _NW
    chunk, k = next(
        (cc, kk)
        for cc in range(128, 7, -8)
        for kk in (5, 4, 3, 2, 1)
        if epw % cc == 0 and (epw // cc) % (2 * kk) == 0
    )
    nchunk = epw // chunk
    nsc = nchunk // k
    js = edge_index[0].reshape(_NW, nchunk, 1, chunk)
    jd = (edge_index[1] + n).reshape(_NW, nchunk, 1, chunk)
    j3 = jnp.concatenate([js, jd], axis=2).reshape(_NW, 2 * nchunk, chunk)

    g = _sc_gather_add(table, j3, nsc, k, chunk)

    w35 = jnp.concatenate([W3, W5], axis=1)
    b35 = jnp.concatenate([b3, b5])
    w46 = jnp.zeros((2 * hdim, 8), jnp.float32)
    w46 = w46.at[:hdim, 0:1].set(W4).at[hdim:, 1 : 1 + c].set(W6)
    b8 = jnp.zeros((8,), jnp.float32).at[0].set(b4[0]).at[1 : 1 + c].set(b6)

    block_e = 2560
    ef_t, o8_t = _edge_mlp(g, W2, b2, w35, b35, w46, b8, block_e)

    existence = o8_t[0]
    assignment = o8_t[1 : 1 + c].T
    return (existence, assignment, ef_t.T)


# speedup vs baseline: 4.5529x; 1.0778x over previous
import functools

import jax
import jax.numpy as jnp
from jax import lax
from jax.experimental import pallas as pl
from jax.experimental.pallas import tpu as pltpu
from jax.experimental.pallas import tpu_sc as plsc

_NUM_CORES = 2
_NUM_SUBCORES = 16
_NW = _NUM_CORES * _NUM_SUBCORES


def _node_projection(x, w1a, w1b, b1):
    n, d = x.shape
    h = w1a.shape[1]

    def body(x_ref, wa_ref, wb_ref, b1_ref, t_ref):
        xv = x_ref[...]
        t_ref[0:n, :] = (
            jnp.dot(xv, wa_ref[...], preferred_element_type=jnp.float32)
            + b1_ref[...]
        )
        t_ref[n : 2 * n, :] = jnp.dot(
            xv, wb_ref[...], preferred_element_type=jnp.float32
        )

    return pl.pallas_call(
        body,
        out_shape=jax.ShapeDtypeStruct((2 * n, h), jnp.float32),
    )(x, w1a, w1b, b1.reshape(1, h))


def _sc_gather_add(table, j3, nsc, k, chunk):
    h = table.shape[1]
    epw = nsc * k * chunk
    rows_total = _NW * epw
    scrows = k * chunk

    mesh = plsc.VectorSubcoreMesh(core_axis_name="c", subcore_axis_name="s")

    @functools.partial(
        pl.kernel,
        mesh=mesh,
        out_type=jax.ShapeDtypeStruct((rows_total, h), jnp.float32),
        scratch_types=[
            pltpu.VMEM((2 * nsc * k, chunk), jnp.int32),
            pltpu.VMEM((2, scrows, h), jnp.float32),
            pltpu.SemaphoreType.DMA,
            pltpu.SemaphoreType.DMA,
            pltpu.SemaphoreType.DMA,
            pltpu.SemaphoreType.DMA,
        ],
    )
    def gather_kernel(t_hbm, j_hbm, g_hbm, idx_v, rows_v, g0, g1, w0, w1):
        gsem = (g0, g1)
        wsem = (w0, w1)
        wid = lax.axis_index("s") * _NUM_CORES + lax.axis_index("c")
        pltpu.sync_copy(j_hbm.at[wid], idx_v)
        base = wid * epw

        def issue(sc_idx, slot, phase):
            for q in range(k):
                ci = (sc_idx * k + q) * 2 + phase
                pltpu.async_copy(
                    t_hbm.at[idx_v.at[ci]],
                    rows_v.at[slot].at[pl.ds(q * chunk, chunk)],
                    gsem[slot],
                    add=(phase == 1),
                )

        def drain(sem, slot):
            pltpu.make_async_copy(
                t_hbm.at[pl.ds(0, scrows)], rows_v.at[slot], sem
            ).wait()

        issue(0, 0, 0)
        issue(1, 1, 0)

        def outer(i0, carry):
            for b in range(2):
                i = i0 * 2 + b
                drain(gsem[b], b)
                issue(i, b, 1)
                drain(gsem[b], b)
                pltpu.async_copy(
                    rows_v.at[b],
                    g_hbm.at[pl.ds(base + i * scrows, scrows)],
                    wsem[b],
                )

                @pl.when(i + 2 < nsc)
                def _():
                    drain(wsem[b], b)
                    issue(i + 2, b, 0)

            return carry

        lax.fori_loop(0, nsc // 2, outer, 0)
        drain(wsem[0], 0)
        drain(wsem[1], 1)

    return gather_kernel(table, j3)


def _edge_mlp_part(gp, w2t, b2c, w35t, b35c, w46t, b8c, block_e,
                   ef_buf, o8_buf, part):
    ep, h = gp.shape
    f, e = ef_buf.shape
    nblk = ep // block_e
    off = part * nblk
    dn_rt = (((1,), (1,)), ((), ()))
    dn_std = (((1,), (0,)), ((), ()))

    def body(g_ref, w2t_ref, b2_ref, w35t_ref, b35_ref, w46t_ref,
             b8_ref, efb_ref, o8b_ref, eft_ref, o8t_ref):
        del efb_ref, o8b_ref
        hid = jnp.maximum(g_ref[...], 0.0)
        ef_t = (
            lax.dot_general(w2t_ref[...], hid, dn_rt,
                            preferred_element_type=jnp.float32)
            + b2_ref[...]
        )
        eft_ref[...] = ef_t
        u_t = jnp.maximum(
            lax.dot_general(w35t_ref[...], ef_t, dn_std,
                            preferred_element_type=jnp.float32)
            + b35_ref[...],
            0.0,
        )
        o8t_ref[...] = (
            lax.dot_general(w46t_ref[...], u_t, dn_std,
                            preferred_element_type=jnp.float32)
            + b8_ref[...]
        )

    full = lambda shape: pl.BlockSpec(shape, lambda i: (0, 0))
    return pl.pallas_call(
        body,
        grid=(nblk,),
        in_specs=[
            pl.BlockSpec((block_e, h), lambda i: (i, 0)),
            full(w2t.shape),
            full((f, 1)),
            full(w35t.shape),
            full((2 * h, 1)),
            full((8, 2 * h)),
            full((8, 1)),
            pl.BlockSpec(memory_space=pl.ANY),
            pl.BlockSpec(memory_space=pl.ANY),
        ],
        out_specs=[
            pl.BlockSpec((f, block_e), lambda i: (0, i + off)),
            pl.BlockSpec((8, block_e), lambda i: (0, i + off)),
        ],
        out_shape=[
            jax.ShapeDtypeStruct((f, e), jnp.float32),
            jax.ShapeDtypeStruct((8, e), jnp.float32),
        ],
        input_output_aliases={7: 0, 8: 1},
    )(gp, w2t, b2c, w35t, b35c, w46t, b8c, ef_buf, o8_buf)


def kernel(node_features, edge_index, W1, b1, W2, b2, W3, b3, W4, b4,
           W5, b5, W6, b6):
    n, d = node_features.shape
    e = edge_index.shape[1]
    hdim = W1.shape[1]
    f = W2.shape[1]
    c = W6.shape[1]

    table = _node_projection(node_features, W1[:d], W1[d:], b1)

    nparts = 5 if e % (5 * _NW) == 0 else 1
    epp = e // nparts
    epw = epp // _NW
    chunk, k = next(
        (cc, kk)
        for cc in range(128, 7, -8)
        for kk in (5, 4, 3, 2, 1)
        if epw % cc == 0 and (epw // cc) % (2 * kk) == 0
    )
    nchunk = epw // chunk
    nsc = nchunk // k
    js = edge_index[0].reshape(nparts, _NW, nchunk, 1, chunk)
    jd = (edge_index[1] + n).reshape(nparts, _NW, nchunk, 1, chunk)
    j3 = jnp.concatenate([js, jd], axis=3)
    j3 = j3.reshape(nparts, _NW, 2 * nchunk, chunk)

    w35 = jnp.concatenate([W3, W5], axis=1)
    b35 = jnp.concatenate([b3, b5])
    w46 = jnp.zeros((2 * hdim, 8), jnp.float32)
    w46 = w46.at[:hdim, 0:1].set(W4).at[hdim:, 1 : 1 + c].set(W6)
    b8 = jnp.zeros((8,), jnp.float32).at[0].set(b4[0]).at[1 : 1 + c].set(b6)

    block_e = 2560 if epp % 2560 == 0 else epp
    ef_buf = jnp.zeros((f, e), jnp.float32)
    o8_buf = jnp.zeros((8, e), jnp.float32)
    w2t = W2.T
    b2c = b2.reshape(f, 1)
    w35t = w35.T
    b35c = b35.reshape(-1, 1)
    w46t = w46.T
    b8c = b8.reshape(8, 1)
    for p in range(nparts):
        gp = _sc_gather_add(table, j3[p], nsc, k, chunk)
        ef_buf, o8_buf = _edge_mlp_part(
            gp, w2t, b2c, w35t, b35c, w46t, b8c, block_e,
            ef_buf, o8_buf, p)

    existence = o8_buf[0]
    assignment = o8_buf[1 : 1 + c].T
    return (existence, assignment, ef_buf.T)


# speedup vs baseline: 4.8631x; 1.0681x over previous
import functools

import jax
import jax.numpy as jnp
from jax import lax
from jax.experimental import pallas as pl
from jax.experimental.pallas import tpu as pltpu
from jax.experimental.pallas import tpu_sc as plsc

_NUM_CORES = 2
_NUM_SUBCORES = 16
_NW = _NUM_CORES * _NUM_SUBCORES


def _node_projection(x, w1a, w1b, b1):
    n, d = x.shape
    h = w1a.shape[1]

    def body(x_ref, wa_ref, wb_ref, b1_ref, t_ref):
        xv = x_ref[...]
        t_ref[0:n, :] = (
            jnp.dot(xv, wa_ref[...], preferred_element_type=jnp.float32)
            + b1_ref[...]
        )
        t_ref[n : 2 * n, :] = jnp.dot(
            xv, wb_ref[...], preferred_element_type=jnp.float32
        )

    return pl.pallas_call(
        body,
        out_shape=jax.ShapeDtypeStruct((2 * n, h), jnp.float32),
    )(x, w1a, w1b, b1.reshape(1, h))


def _sc_gather_add(table, j3, nsc, k, chunk):
    h = table.shape[1]
    epw = nsc * k * chunk
    rows_total = _NW * epw
    scrows = k * chunk

    mesh = plsc.VectorSubcoreMesh(core_axis_name="c", subcore_axis_name="s")

    @functools.partial(
        pl.kernel,
        mesh=mesh,
        out_type=jax.ShapeDtypeStruct((rows_total, h), jnp.float32),
        scratch_types=[
            pltpu.VMEM((2 * nsc * k, chunk), jnp.int32),
            pltpu.VMEM((2, scrows, h), jnp.float32),
            pltpu.SemaphoreType.DMA,
            pltpu.SemaphoreType.DMA,
            pltpu.SemaphoreType.DMA,
            pltpu.SemaphoreType.DMA,
        ],
    )
    def gather_kernel(t_hbm, j_hbm, g_hbm, idx_v, rows_v, g0, g1, w0, w1):
        gsem = (g0, g1)
        wsem = (w0, w1)
        wid = lax.axis_index("s") * _NUM_CORES + lax.axis_index("c")
        pltpu.sync_copy(j_hbm.at[wid], idx_v)
        base = wid * epw

        def issue(sc_idx, slot, phase):
            for q in range(k):
                ci = (sc_idx * k + q) * 2 + phase
                pltpu.async_copy(
                    t_hbm.at[idx_v.at[ci]],
                    rows_v.at[slot].at[pl.ds(q * chunk, chunk)],
                    gsem[slot],
                    add=(phase == 1),
                )

        def drain(sem, slot):
            pltpu.make_async_copy(
                t_hbm.at[pl.ds(0, scrows)], rows_v.at[slot], sem
            ).wait()

        issue(0, 0, 0)
        issue(1, 1, 0)

        def outer(i0, carry):
            for b in range(2):
                i = i0 * 2 + b
                drain(gsem[b], b)
                issue(i, b, 1)
                drain(gsem[b], b)
                pltpu.async_copy(
                    rows_v.at[b],
                    g_hbm.at[pl.ds(base + i * scrows, scrows)],
                    wsem[b],
                )

                @pl.when(i + 2 < nsc)
                def _():
                    drain(wsem[b], b)
                    issue(i + 2, b, 0)

            return carry

        lax.fori_loop(0, nsc // 2, outer, 0)
        drain(wsem[0], 0)
        drain(wsem[1], 1)

    return gather_kernel(table, j3)


def _edge_mlp_part(gp, w2t, b2c, w35t, b35c, w46t, b8c, block_e,
                   ef_buf, o8_buf, part, e):
    ep, h = gp.shape
    f = w2t.shape[0]
    nblk = ep // block_e
    off = part * nblk
    dn_rt = (((1,), (1,)), ((), ()))
    dn_std = (((1,), (0,)), ((), ()))

    def body(g_ref, w2t_ref, b2_ref, w35t_ref, b35_ref, w46t_ref,
             b8_ref, *refs):
        eft_ref, o8t_ref = refs[-2], refs[-1]
        hid = jnp.maximum(g_ref[...], 0.0)
        ef_t = (
            lax.dot_general(w2t_ref[...], hid, dn_rt,
                            preferred_element_type=jnp.float32)
            + b2_ref[...]
        )
        eft_ref[...] = ef_t
        u_t = jnp.maximum(
            lax.dot_general(w35t_ref[...], ef_t, dn_std,
                            preferred_element_type=jnp.float32)
            + b35_ref[...],
            0.0,
        )
        o8t_ref[...] = (
            lax.dot_general(w46t_ref[...], u_t, dn_std,
                            preferred_element_type=jnp.float32)
            + b8_ref[...]
        )

    full = lambda shape: pl.BlockSpec(shape, lambda i: (0, 0))
    in_specs = [
        pl.BlockSpec((block_e, h), lambda i: (i, 0)),
        full(w2t.shape),
        full((f, 1)),
        full(w35t.shape),
        full((2 * h, 1)),
        full((8, 2 * h)),
        full((8, 1)),
    ]
    args = [gp, w2t, b2c, w35t, b35c, w46t, b8c]
    aliases = {}
    if ef_buf is not None:
        in_specs += [pl.BlockSpec(memory_space=pl.ANY),
                     pl.BlockSpec(memory_space=pl.ANY)]
        args += [ef_buf, o8_buf]
        aliases = {7: 0, 8: 1}
    return pl.pallas_call(
        body,
        grid=(nblk,),
        in_specs=in_specs,
        out_specs=[
            pl.BlockSpec((f, block_e), lambda i: (0, i + off)),
            pl.BlockSpec((8, block_e), lambda i: (0, i + off)),
        ],
        out_shape=[
            jax.ShapeDtypeStruct((f, e), jnp.float32),
            jax.ShapeDtypeStruct((8, e), jnp.float32),
        ],
        input_output_aliases=aliases,
    )(*args)


def kernel(node_features, edge_index, W1, b1, W2, b2, W3, b3, W4, b4,
           W5, b5, W6, b6):
    n, d = node_features.shape
    e = edge_index.shape[1]
    hdim = W1.shape[1]
    f = W2.shape[1]
    c = W6.shape[1]

    table = _node_projection(node_features, W1[:d], W1[d:], b1)

    nparts = 5 if e % (5 * _NW) == 0 else 1
    epp = e // nparts
    epw = epp // _NW
    chunk, k = next(
        (cc, kk)
        for cc in range(128, 7, -8)
        for kk in (5, 4, 3, 2, 1)
        if epw % cc == 0 and (epw // cc) % (2 * kk) == 0
    )
    nchunk = epw // chunk
    nsc = nchunk // k
    js = edge_index[0].reshape(nparts, _NW, nchunk, 1, chunk)
    jd = (edge_index[1] + n).reshape(nparts, _NW, nchunk, 1, chunk)
    j3 = jnp.concatenate([js, jd], axis=3)
    j3 = j3.reshape(nparts, _NW, 2 * nchunk, chunk)

    w35 = jnp.concatenate([W3, W5], axis=1)
    b35 = jnp.concatenate([b3, b5])
    w46 = jnp.zeros((2 * hdim, 8), jnp.float32)
    w46 = w46.at[:hdim, 0:1].set(W4).at[hdim:, 1 : 1 + c].set(W6)
    b8 = jnp.zeros((8,), jnp.float32).at[0].set(b4[0]).at[1 : 1 + c].set(b6)

    block_e = 2560 if epp % 2560 == 0 else epp
    ef_buf = o8_buf = None
    w2t = W2.T
    b2c = b2.reshape(f, 1)
    w35t = w35.T
    b35c = b35.reshape(-1, 1)
    w46t = w46.T
    b8c = b8.reshape(8, 1)
    for p in range(nparts):
        gp = _sc_gather_add(table, j3[p], nsc, k, chunk)
        ef_buf, o8_buf = _edge_mlp_part(
            gp, w2t, b2c, w35t, b35c, w46t, b8c, block_e,
            ef_buf, o8_buf, p, e)

    existence = o8_buf[0]
    assignment = o8_buf[1 : 1 + c].T
    return (existence, assignment, ef_buf.T)


# speedup vs baseline: 5.0139x; 1.0310x over previous
import functools

import jax
import jax.numpy as jnp
from jax import lax
from jax.experimental import pallas as pl
from jax.experimental.pallas import tpu as pltpu
from jax.experimental.pallas import tpu_sc as plsc

_NUM_CORES = 2
_NUM_SUBCORES = 16
_NW = _NUM_CORES * _NUM_SUBCORES


def _node_projection(x, w1a, w1b, b1):
    n, d = x.shape
    h = w1a.shape[1]

    def body(x_ref, wa_ref, wb_ref, b1_ref, t_ref):
        xv = x_ref[...]
        t_ref[0:n, :] = (
            jnp.dot(xv, wa_ref[...], preferred_element_type=jnp.float32)
            + b1_ref[...]
        )
        t_ref[n : 2 * n, :] = jnp.dot(
            xv, wb_ref[...], preferred_element_type=jnp.float32
        )

    return pl.pallas_call(
        body,
        out_shape=jax.ShapeDtypeStruct((2 * n, h), jnp.float32),
    )(x, w1a, w1b, b1.reshape(1, h))


def _sc_gather_add(table, j3, nsc, k, chunk):
    h = table.shape[1]
    epw = nsc * k * chunk
    rows_total = _NW * epw
    scrows = k * chunk

    mesh = plsc.VectorSubcoreMesh(core_axis_name="c", subcore_axis_name="s")

    @functools.partial(
        pl.kernel,
        mesh=mesh,
        out_type=jax.ShapeDtypeStruct((rows_total, h), jnp.float32),
        scratch_types=[
            pltpu.VMEM((2 * nsc * k, chunk), jnp.int32),
            pltpu.VMEM((2, scrows, h), jnp.float32),
            pltpu.SemaphoreType.DMA,
            pltpu.SemaphoreType.DMA,
            pltpu.SemaphoreType.DMA,
            pltpu.SemaphoreType.DMA,
        ],
    )
    def gather_kernel(t_hbm, j_hbm, g_hbm, idx_v, rows_v, g0, g1, w0, w1):
        gsem = (g0, g1)
        wsem = (w0, w1)
        wid = lax.axis_index("s") * _NUM_CORES + lax.axis_index("c")
        pltpu.sync_copy(j_hbm.at[wid], idx_v)
        base = wid * epw

        def issue(sc_idx, slot, phase):
            for q in range(k):
                ci = (sc_idx * k + q) * 2 + phase
                pltpu.async_copy(
                    t_hbm.at[idx_v.at[ci]],
                    rows_v.at[slot].at[pl.ds(q * chunk, chunk)],
                    gsem[slot],
                    add=(phase == 1),
                )

        def drain(sem, slot):
            pltpu.make_async_copy(
                t_hbm.at[pl.ds(0, scrows)], rows_v.at[slot], sem
            ).wait()

        def process(i, b, refill):
            drain(gsem[b], b)
            issue(i, b, 1)
            drain(gsem[b], b)
            pltpu.async_copy(
                rows_v.at[b],
                g_hbm.at[pl.ds(base + i * scrows, scrows)],
                wsem[b],
            )
            if refill:
                @pl.when(i + 2 < nsc)
                def _():
                    drain(wsem[b], b)
                    issue(i + 2, b, 0)

        issue(0, 0, 0)
        issue(1, 1, 0)

        def outer(i0, carry):
            for b in range(2):
                process(i0 * 2 + b, b, refill=True)
            return carry

        lax.fori_loop(0, nsc // 2, outer, 0)
        if nsc % 2:
            process(nsc - 1, (nsc - 1) % 2, refill=False)
        drain(wsem[0], 0)
        drain(wsem[1], 1)

    return gather_kernel(table, j3)


def _edge_mlp_part(gp, w2t, b2c, w35t, b35c, w46t, b8c, block_e,
                   ef_buf, o8_buf, part, e):
    ep, h = gp.shape
    f = w2t.shape[0]
    nblk = ep // block_e
    off = part * nblk
    dn_rt = (((1,), (1,)), ((), ()))
    dn_std = (((1,), (0,)), ((), ()))

    def body(g_ref, w2t_ref, b2_ref, w35t_ref, b35_ref, w46t_ref,
             b8_ref, *refs):
        eft_ref, o8t_ref = refs[-2], refs[-1]
        hid = jnp.maximum(g_ref[...], 0.0)
        ef_t = (
            lax.dot_general(w2t_ref[...], hid, dn_rt,
                            preferred_element_type=jnp.float32)
            + b2_ref[...]
        )
        eft_ref[...] = ef_t
        u_t = jnp.maximum(
            lax.dot_general(w35t_ref[...], ef_t, dn_std,
                            preferred_element_type=jnp.float32)
            + b35_ref[...],
            0.0,
        )
        o8t_ref[...] = (
            lax.dot_general(w46t_ref[...], u_t, dn_std,
                            preferred_element_type=jnp.float32)
            + b8_ref[...]
        )

    full = lambda shape: pl.BlockSpec(shape, lambda i: (0, 0))
    in_specs = [
        pl.BlockSpec((block_e, h), lambda i: (i, 0)),
        full(w2t.shape),
        full((f, 1)),
        full(w35t.shape),
        full((2 * h, 1)),
        full((8, 2 * h)),
        full((8, 1)),
    ]
    args = [gp, w2t, b2c, w35t, b35c, w46t, b8c]
    aliases = {}
    if ef_buf is not None:
        in_specs += [pl.BlockSpec(memory_space=pl.ANY),
                     pl.BlockSpec(memory_space=pl.ANY)]
        args += [ef_buf, o8_buf]
        aliases = {7: 0, 8: 1}
    return pl.pallas_call(
        body,
        grid=(nblk,),
        in_specs=in_specs,
        out_specs=[
            pl.BlockSpec((f, block_e), lambda i: (0, i + off)),
            pl.BlockSpec((8, block_e), lambda i: (0, i + off)),
        ],
        out_shape=[
            jax.ShapeDtypeStruct((f, e), jnp.float32),
            jax.ShapeDtypeStruct((8, e), jnp.float32),
        ],
        input_output_aliases=aliases,
    )(*args)


def kernel(node_features, edge_index, W1, b1, W2, b2, W3, b3, W4, b4,
           W5, b5, W6, b6):
    n, d = node_features.shape
    e = edge_index.shape[1]
    hdim = W1.shape[1]
    f = W2.shape[1]
    c = W6.shape[1]

    table = _node_projection(node_features, W1[:d], W1[d:], b1)

    nparts = 5 if e % (5 * _NW) == 0 else 1
    epp = e // nparts
    epw = epp // _NW
    best = None
    for cc in range(128, 7, -8):
        if epw % cc:
            continue
        nch = epw // cc
        for kk in range(min(nch, 12), 0, -1):
            if nch % kk or nch // kk < 2:
                continue
            if 2 * cc * kk * hdim + 2 * nch * cc > 110000:
                continue
            if best is None or cc * kk > best[0] * best[1]:
                best = (cc, kk)
    chunk, k = best
    nchunk = epw // chunk
    nsc = nchunk // k
    js = edge_index[0].reshape(nparts, _NW, nchunk, 1, chunk)
    jd = (edge_index[1] + n).reshape(nparts, _NW, nchunk, 1, chunk)
    j3 = jnp.concatenate([js, jd], axis=3)
    j3 = j3.reshape(nparts, _NW, 2 * nchunk, chunk)

    w35 = jnp.concatenate([W3, W5], axis=1)
    b35 = jnp.concatenate([b3, b5])
    w46 = jnp.zeros((2 * hdim, 8), jnp.float32)
    w46 = w46.at[:hdim, 0:1].set(W4).at[hdim:, 1 : 1 + c].set(W6)
    b8 = jnp.zeros((8,), jnp.float32).at[0].set(b4[0]).at[1 : 1 + c].set(b6)

    block_e = 2560 if epp % 2560 == 0 else epp
    ef_buf = o8_buf = None
    w2t = W2.T
    b2c = b2.reshape(f, 1)
    w35t = w35.T
    b35c = b35.reshape(-1, 1)
    w46t = w46.T
    b8c = b8.reshape(8, 1)
    for p in range(nparts):
        gp = _sc_gather_add(table, j3[p], nsc, k, chunk)
        ef_buf, o8_buf = _edge_mlp_part(
            gp, w2t, b2c, w35t, b35c, w46t, b8c, block_e,
            ef_buf, o8_buf, p, e)

    existence = o8_buf[0]
    assignment = o8_buf[1 : 1 + c].T
    return (existence, assignment, ef_buf.T)


# speedup vs baseline: 5.0452x; 1.0062x over previous
import functools

import jax
import jax.numpy as jnp
from jax import lax
from jax.experimental import pallas as pl
from jax.experimental.pallas import tpu as pltpu
from jax.experimental.pallas import tpu_sc as plsc

_NUM_CORES = 2
_NUM_SUBCORES = 16
_NW = _NUM_CORES * _NUM_SUBCORES


def _node_projection(x, w1a, w1b, b1):
    n, d = x.shape
    h = w1a.shape[1]

    def body(x_ref, wa_ref, wb_ref, b1_ref, t_ref):
        xv = x_ref[...]
        t_ref[0:n, :] = (
            jnp.dot(xv, wa_ref[...], preferred_element_type=jnp.float32)
            + b1_ref[...]
        )
        t_ref[n : 2 * n, :] = jnp.dot(
            xv, wb_ref[...], preferred_element_type=jnp.float32
        )

    return pl.pallas_call(
        body,
        out_shape=jax.ShapeDtypeStruct((2 * n, h), jnp.float32),
    )(x, w1a, w1b, b1.reshape(1, h))


def _sc_gather_add(table, j3, nsc, k, chunk):
    h = table.shape[1]
    epw = nsc * k * chunk
    rows_total = _NW * epw
    scrows = k * chunk

    mesh = plsc.VectorSubcoreMesh(core_axis_name="c", subcore_axis_name="s")

    @functools.partial(
        pl.kernel,
        mesh=mesh,
        out_type=jax.ShapeDtypeStruct((rows_total, h), jnp.float32),
        scratch_types=[
            pltpu.VMEM((2 * nsc * k, chunk), jnp.int32),
            pltpu.VMEM((2, scrows, h), jnp.float32),
            pltpu.SemaphoreType.DMA,
            pltpu.SemaphoreType.DMA,
            pltpu.SemaphoreType.DMA,
            pltpu.SemaphoreType.DMA,
        ],
    )
    def gather_kernel(t_hbm, j_hbm, g_hbm, idx_v, rows_v, g0, g1, w0, w1):
        gsem = (g0, g1)
        wsem = (w0, w1)
        wid = lax.axis_index("s") * _NUM_CORES + lax.axis_index("c")
        pltpu.sync_copy(j_hbm.at[wid], idx_v)
        base = wid * epw

        def issue(sc_idx, slot, phase):
            for q in range(k):
                ci = (sc_idx * k + q) * 2 + phase
                pltpu.async_copy(
                    t_hbm.at[idx_v.at[ci]],
                    rows_v.at[slot].at[pl.ds(q * chunk, chunk)],
                    gsem[slot],
                    add=(phase == 1),
                )

        def drain(sem, slot):
            pltpu.make_async_copy(
                t_hbm.at[pl.ds(0, scrows)], rows_v.at[slot], sem
            ).wait()

        def process(i, b, refill):
            drain(gsem[b], b)
            issue(i, b, 1)
            drain(gsem[b], b)
            pltpu.async_copy(
                rows_v.at[b],
                g_hbm.at[pl.ds(base + i * scrows, scrows)],
                wsem[b],
            )
            if refill:
                @pl.when(i + 2 < nsc)
                def _():
                    drain(wsem[b], b)
                    issue(i + 2, b, 0)

        issue(0, 0, 0)
        issue(1, 1, 0)

        def outer(i0, carry):
            for b in range(2):
                process(i0 * 2 + b, b, refill=True)
            return carry

        lax.fori_loop(0, nsc // 2, outer, 0)
        if nsc % 2:
            process(nsc - 1, (nsc - 1) % 2, refill=False)
        drain(wsem[0], 0)
        drain(wsem[1], 1)

    return gather_kernel(table, j3)


def _edge_mlp_part(gp, w2t, b2c, w35t, b35c, w46t, b8c, block_e,
                   ef_buf, o8_buf, part, e):
    ep, h = gp.shape
    f = w2t.shape[0]
    nblk = ep // block_e
    off = part * nblk
    dn_rt = (((1,), (1,)), ((), ()))
    dn_std = (((1,), (0,)), ((), ()))

    def body(g_ref, w2t_ref, b2_ref, w35t_ref, b35_ref, w46t_ref,
             b8_ref, *refs):
        eft_ref, o8t_ref = refs[-2], refs[-1]
        hid = jnp.maximum(g_ref[...], 0.0)
        ef_t = (
            lax.dot_general(w2t_ref[...], hid, dn_rt,
                            preferred_element_type=jnp.float32)
            + b2_ref[...]
        )
        eft_ref[...] = ef_t
        u_t = jnp.maximum(
            lax.dot_general(w35t_ref[...], ef_t, dn_std,
                            preferred_element_type=jnp.float32)
            + b35_ref[...],
            0.0,
        )
        o8t_ref[...] = (
            lax.dot_general(w46t_ref[...], u_t, dn_std,
                            preferred_element_type=jnp.float32)
            + b8_ref[...]
        )

    full = lambda shape: pl.BlockSpec(shape, lambda i: (0, 0))
    in_specs = [
        pl.BlockSpec((block_e, h), lambda i: (i, 0)),
        full(w2t.shape),
        full((f, 1)),
        full(w35t.shape),
        full((2 * h, 1)),
        full((8, 2 * h)),
        full((8, 1)),
    ]
    args = [gp, w2t, b2c, w35t, b35c, w46t, b8c]
    aliases = {}
    if ef_buf is not None:
        in_specs += [pl.BlockSpec(memory_space=pl.ANY),
                     pl.BlockSpec(memory_space=pl.ANY)]
        args += [ef_buf, o8_buf]
        aliases = {7: 0, 8: 1}
    return pl.pallas_call(
        body,
        grid=(nblk,),
        in_specs=in_specs,
        out_specs=[
            pl.BlockSpec((f, block_e), lambda i: (0, i + off)),
            pl.BlockSpec((8, block_e), lambda i: (0, i + off)),
        ],
        out_shape=[
            jax.ShapeDtypeStruct((f, e), jnp.float32),
            jax.ShapeDtypeStruct((8, e), jnp.float32),
        ],
        input_output_aliases=aliases,
    )(*args)


def kernel(node_features, edge_index, W1, b1, W2, b2, W3, b3, W4, b4,
           W5, b5, W6, b6):
    n, d = node_features.shape
    e = edge_index.shape[1]
    hdim = W1.shape[1]
    f = W2.shape[1]
    c = W6.shape[1]

    table = _node_projection(node_features, W1[:d], W1[d:], b1)

    nparts = 5 if e % (5 * _NW) == 0 else 1
    epp = e // nparts
    epw = epp // _NW
    best = None
    for cc in range(128, 7, -8):
        if epw % cc:
            continue
        nch = epw // cc
        for kk in range(min(nch, 12), 0, -1):
            if nch % kk or nch // kk < 2:
                continue
            if 2 * cc * kk * hdim + 2 * nch * cc > 110000:
                continue
            if best is None or cc * kk > best[0] * best[1]:
                best = (cc, kk)
    chunk, k = best
    nchunk = epw // chunk
    nsc = nchunk // k
    js = edge_index[0].reshape(nparts, _NW, nchunk, 1, chunk)
    jd = (edge_index[1] + n).reshape(nparts, _NW, nchunk, 1, chunk)
    j3 = jnp.concatenate([js, jd], axis=3)
    j3 = j3.reshape(nparts, _NW, 2 * nchunk, chunk)

    w35 = jnp.concatenate([W3, W5], axis=1)
    b35 = jnp.concatenate([b3, b5])
    w46 = jnp.zeros((2 * hdim, 8), jnp.float32)
    w46 = w46.at[:hdim, 0:1].set(W4).at[hdim:, 1 : 1 + c].set(W6)
    b8 = jnp.zeros((8,), jnp.float32).at[0].set(b4[0]).at[1 : 1 + c].set(b6)

    block_e = 3200 if epp % 3200 == 0 else epp
    ef_buf = o8_buf = None
    w2t = W2.T
    b2c = b2.reshape(f, 1)
    w35t = w35.T
    b35c = b35.reshape(-1, 1)
    w46t = w46.T
    b8c = b8.reshape(8, 1)
    for p in range(nparts):
        gp = _sc_gather_add(table, j3[p], nsc, k, chunk)
        ef_buf, o8_buf = _edge_mlp_part(
            gp, w2t, b2c, w35t, b35c, w46t, b8c, block_e,
            ef_buf, o8_buf, p, e)

    existence = o8_buf[0]
    assignment = o8_buf[1 : 1 + c].T
    return (existence, assignment, ef_buf.T)
